# Initial kernel scaffold; baseline (speedup 1.0000x reference)
import jax
import jax.numpy as jnp
from jax.experimental import pallas as pl


def kernel(nodes_batch, adj, raw_features, W1, W2):
    raise NotImplementedError("write your pallas kernel here")


# speedup vs baseline: 9.9962x; 9.9962x over previous
import functools

import jax
import jax.numpy as jnp
from jax import lax
from jax.experimental import pallas as pl
from jax.experimental.pallas import tpu as pltpu
from jax.experimental.pallas import tpu_sc as plsc

N = 10000
DEG = 32
D = 128
OUT = 128
B = 4096
S1 = 16
S2 = 4

NC, NS, L = 2, 16, 16
NW = NC * NS
NPAD = 10240
NODES_PER_W = NPAD // NW
CH_A = 32
B_PER_W = B // NW
CH_C = 8

_MESH = plsc.VectorSubcoreMesh(
    core_axis_name="c", subcore_axis_name="s", num_cores=NC, num_subcores=NS
)


@functools.partial(
    pl.kernel,
    out_type=jax.ShapeDtypeStruct((NPAD, D), jnp.float32),
    mesh=_MESH,
    scratch_types=[
        pltpu.VMEM((CH_A * S2,), jnp.int32),
        pltpu.VMEM((CH_A * S2, D), jnp.float32),
        pltpu.VMEM((CH_A, D), jnp.float32),
        pltpu.SemaphoreType.DMA,
    ],
)
def _preagg(idx_hbm, feat_hbm, out_hbm, idx_v, rows_v, acc_v, sem):
    wid = lax.axis_index("s") * NC + lax.axis_index("c")
    base = wid * NODES_PER_W

    def chunk(ci, carry):
        nb = base + ci * CH_A
        pltpu.sync_copy(idx_hbm.at[pl.ds(nb * S2, CH_A * S2)], idx_v)
        pltpu.async_copy(feat_hbm.at[idx_v], rows_v, sem).wait()

        def node(i, c2):
            for k in range(D // L):
                s = rows_v[i * S2, pl.ds(k * L, L)]
                for p in range(1, S2):
                    s = s + rows_v[i * S2 + p, pl.ds(k * L, L)]
                acc_v[i, pl.ds(k * L, L)] = s * (1.0 / S2)
            return c2

        lax.fori_loop(0, CH_A, node, 0)
        pltpu.sync_copy(acc_v, out_hbm.at[pl.ds(nb, CH_A)])
        return carry

    lax.fori_loop(0, NODES_PER_W // CH_A, chunk, 0)


@functools.partial(
    pl.kernel,
    out_type=(
        jax.ShapeDtypeStruct((B, OUT), jnp.float32),
        jax.ShapeDtypeStruct((B, OUT), jnp.float32),
    ),
    mesh=_MESH,
    scratch_types=[
        pltpu.VMEM((B_PER_W,), jnp.int32),
        pltpu.VMEM((B_PER_W, 128), jnp.int32),
        pltpu.VMEM((B_PER_W, OUT), jnp.float32),
        pltpu.VMEM((CH_C * S1,), jnp.int32),
        pltpu.VMEM((CH_C * S1, OUT), jnp.float32),
        pltpu.VMEM((CH_C, OUT), jnp.float32),
        pltpu.SemaphoreType.DMA,
        pltpu.SemaphoreType.DMA,
    ],
)
def _batch(nb_hbm, adj_hbm, h_hbm, hself_out, agg_out,
           nb_v, adjr_v, hself_v, nidx_v, nrows_v, agg_v, sem0, sem1):
    wid = lax.axis_index("s") * NC + lax.axis_index("c")
    base = wid * B_PER_W
    pltpu.sync_copy(nb_hbm.at[pl.ds(base, B_PER_W)], nb_v)
    cp_adj = pltpu.async_copy(adj_hbm.at[nb_v], adjr_v, sem0)
    cp_self = pltpu.async_copy(h_hbm.at[nb_v], hself_v, sem1)
    cp_adj.wait()
    cp_self.wait()
    pltpu.sync_copy(hself_v, hself_out.at[pl.ds(base, B_PER_W)])

    def chunk(ci, carry):
        def build(b, c2):
            nidx_v[pl.ds(b * S1, S1)] = adjr_v[ci * CH_C + b, pl.ds(0, S1)]
            return c2

        lax.fori_loop(0, CH_C, build, 0)
        pltpu.async_copy(h_hbm.at[nidx_v], nrows_v, sem0).wait()

        def bacc(b, c2):
            for k in range(OUT // L):
                s = nrows_v[b * S1, pl.ds(k * L, L)]
                for p in range(1, S1):
                    s = s + nrows_v[b * S1 + p, pl.ds(k * L, L)]
                agg_v[b, pl.ds(k * L, L)] = s * (1.0 / S1)
            return c2

        lax.fori_loop(0, CH_C, bacc, 0)
        pltpu.sync_copy(agg_v, agg_out.at[pl.ds(base + ci * CH_C, CH_C)])
        return carry

    lax.fori_loop(0, B_PER_W // CH_C, chunk, 0)


def _mm_body(a_ref, b_ref, wa_ref, wb_ref, o_ref):
    acc = jnp.dot(a_ref[...], wa_ref[...], preferred_element_type=jnp.float32)
    acc = acc + jnp.dot(b_ref[...], wb_ref[...], preferred_element_type=jnp.float32)
    o_ref[...] = jnp.maximum(acc, 0.0)


def _mm_relu(a, b, wa, wb, bm):
    m = a.shape[0]
    return pl.pallas_call(
        _mm_body,
        grid=(m // bm,),
        in_specs=[
            pl.BlockSpec((bm, D), lambda i: (i, 0)),
            pl.BlockSpec((bm, D), lambda i: (i, 0)),
            pl.BlockSpec((D, OUT), lambda i: (0, 0)),
            pl.BlockSpec((D, OUT), lambda i: (0, 0)),
        ],
        out_specs=pl.BlockSpec((bm, OUT), lambda i: (i, 0)),
        out_shape=jax.ShapeDtypeStruct((m, OUT), jnp.float32),
    )(a, b, wa, wb)


def kernel(nodes_batch, adj, raw_features, W1, W2):
    idx_a = jnp.pad(adj[:, :S2].reshape(-1), (0, (NPAD - N) * S2))
    feat_p = jnp.pad(raw_features, ((0, NPAD - N), (0, 0)))
    adj_p = jnp.pad(adj[:, :S1], ((0, 0), (0, 128 - S1)))
    pre_agg = _preagg(idx_a, feat_p)
    h = _mm_relu(feat_p, pre_agg, W1[:D], W1[D:], 512)
    h_self, agg2 = _batch(nodes_batch, adj_p, h)
    return _mm_relu(h_self, agg2, W2[:OUT], W2[OUT:], 512)


# speedup vs baseline: 13.0226x; 1.3028x over previous
import functools

import jax
import jax.numpy as jnp
from jax import lax
from jax.experimental import pallas as pl
from jax.experimental.pallas import tpu as pltpu
from jax.experimental.pallas import tpu_sc as plsc

N = 10000
DEG = 32
D = 128
OUT = 128
B = 4096
S1 = 16
S2 = 4

NC, NS, L = 2, 16, 16
NW = NC * NS
NPAD = 10240
NODES_PER_W = NPAD // NW
CH_A = 32
NCH_A = NODES_PER_W // CH_A
NBUF_A = 4
B_PER_W = B // NW
CH_C = 8
NCH_C = B_PER_W // CH_C
NBUF_C = 3

_MESH = plsc.VectorSubcoreMesh(
    core_axis_name="c", subcore_axis_name="s", num_cores=NC, num_subcores=NS
)


@functools.partial(
    pl.kernel,
    out_type=jax.ShapeDtypeStruct((NPAD, D), jnp.float32),
    mesh=_MESH,
    scratch_types=[
        pltpu.VMEM((NODES_PER_W * S2,), jnp.int32),
    ]
    + [pltpu.VMEM((CH_A * S2, D), jnp.float32) for _ in range(NBUF_A)]
    + [pltpu.VMEM((NODES_PER_W, D), jnp.float32)]
    + [pltpu.SemaphoreType.DMA for _ in range(NBUF_A)],
)
def _preagg(idx_hbm, feat_hbm, out_hbm, idx_v, r0, r1, r2, r3, ob,
            s0, s1, s2, s3):
    rows = (r0, r1, r2, r3)
    sems = (s0, s1, s2, s3)
    wid = lax.axis_index("s") * NC + lax.axis_index("c")
    base = wid * NODES_PER_W
    pltpu.sync_copy(idx_hbm.at[pl.ds(base * S2, NODES_PER_W * S2)], idx_v)

    def issue(ch):
        b = ch % NBUF_A
        src = feat_hbm.at[idx_v.at[pl.ds(ch * CH_A * S2, CH_A * S2)]]
        return pltpu.async_copy(src, rows[b], sems[b])

    cps = {ch: issue(ch) for ch in range(NBUF_A)}
    for ch in range(NCH_A):
        b = ch % NBUF_A
        cps[ch].wait()

        def node(i, c2, _rv=rows[b], _off=ch * CH_A):
            for k in range(D // L):
                s = _rv[i * S2, pl.ds(k * L, L)]
                for p in range(1, S2):
                    s = s + _rv[i * S2 + p, pl.ds(k * L, L)]
                ob[_off + i, pl.ds(k * L, L)] = s * (1.0 / S2)
            return c2

        lax.fori_loop(0, CH_A, node, 0)
        if ch + NBUF_A < NCH_A:
            cps[ch + NBUF_A] = issue(ch + NBUF_A)
    pltpu.sync_copy(ob, out_hbm.at[pl.ds(base, NODES_PER_W)])


@functools.partial(
    pl.kernel,
    out_type=(
        jax.ShapeDtypeStruct((B, OUT), jnp.float32),
        jax.ShapeDtypeStruct((B, OUT), jnp.float32),
    ),
    mesh=_MESH,
    scratch_types=[
        pltpu.VMEM((B_PER_W,), jnp.int32),
        pltpu.VMEM((B_PER_W, 128), jnp.int32),
        pltpu.VMEM((B_PER_W, OUT), jnp.float32),
    ]
    + [pltpu.VMEM((CH_C * S1,), jnp.int32) for _ in range(NBUF_C)]
    + [pltpu.VMEM((CH_C * S1, OUT), jnp.float32) for _ in range(NBUF_C)]
    + [pltpu.VMEM((B_PER_W, OUT), jnp.float32)]
    + [pltpu.SemaphoreType.DMA for _ in range(NBUF_C + 2)],
)
def _batch(nb_hbm, adj_hbm, h_hbm, hself_out, agg_out,
           nb_v, adjr_v, hself_v, i0, i1, i2, r0, r1, r2, aggb,
           s0, s1, s2, sa, sh):
    nidx = (i0, i1, i2)
    nrows = (r0, r1, r2)
    sems = (s0, s1, s2)
    wid = lax.axis_index("s") * NC + lax.axis_index("c")
    base = wid * B_PER_W
    pltpu.sync_copy(nb_hbm.at[pl.ds(base, B_PER_W)], nb_v)
    cp_adj = pltpu.async_copy(adj_hbm.at[nb_v], adjr_v, sa)
    cp_self = pltpu.async_copy(h_hbm.at[nb_v], hself_v, sh)
    cp_adj.wait()

    def issue(ch):
        b = ch % NBUF_C
        _ni = nidx[b]

        def build(j, c2, _off=ch * CH_C):
            _ni[pl.ds(j * S1, S1)] = adjr_v[_off + j, pl.ds(0, S1)]
            return c2

        lax.fori_loop(0, CH_C, build, 0)
        return pltpu.async_copy(h_hbm.at[_ni], nrows[b], sems[b])

    cps = {ch: issue(ch) for ch in range(NBUF_C)}
    for ch in range(NCH_C):
        b = ch % NBUF_C
        cps[ch].wait()

        def bacc(j, c2, _rv=nrows[b], _off=ch * CH_C):
            def kstep(k, c3):
                s = _rv[j * S1, pl.ds(k * L, L)]
                for p in range(1, S1):
                    s = s + _rv[j * S1 + p, pl.ds(k * L, L)]
                aggb[_off + j, pl.ds(k * L, L)] = s * (1.0 / S1)
                return c3

            lax.fori_loop(0, OUT // L, kstep, 0)
            return c2

        lax.fori_loop(0, CH_C, bacc, 0)
        if ch + NBUF_C < NCH_C:
            cps[ch + NBUF_C] = issue(ch + NBUF_C)
    cp_self.wait()
    pltpu.sync_copy(hself_v, hself_out.at[pl.ds(base, B_PER_W)])
    pltpu.sync_copy(aggb, agg_out.at[pl.ds(base, B_PER_W)])


def _mm_body(a_ref, b_ref, wa_ref, wb_ref, o_ref):
    acc = jnp.dot(a_ref[...], wa_ref[...], preferred_element_type=jnp.float32)
    acc = acc + jnp.dot(b_ref[...], wb_ref[...], preferred_element_type=jnp.float32)
    o_ref[...] = jnp.maximum(acc, 0.0)


def _mm_relu(a, b, wa, wb, bm):
    m = a.shape[0]
    return pl.pallas_call(
        _mm_body,
        grid=(m // bm,),
        in_specs=[
            pl.BlockSpec((bm, D), lambda i: (i, 0)),
            pl.BlockSpec((bm, D), lambda i: (i, 0)),
            pl.BlockSpec((D, OUT), lambda i: (0, 0)),
            pl.BlockSpec((D, OUT), lambda i: (0, 0)),
        ],
        out_specs=pl.BlockSpec((bm, OUT), lambda i: (i, 0)),
        out_shape=jax.ShapeDtypeStruct((m, OUT), jnp.float32),
    )(a, b, wa, wb)


def kernel(nodes_batch, adj, raw_features, W1, W2):
    idx_a = jnp.pad(adj[:, :S2].reshape(-1), (0, (NPAD - N) * S2))
    feat_p = jnp.pad(raw_features, ((0, NPAD - N), (0, 0)))
    adj_p = jnp.pad(adj[:, :S1], ((0, 0), (0, 128 - S1)))
    pre_agg = _preagg(idx_a, feat_p)
    h = _mm_relu(feat_p, pre_agg, W1[:D], W1[D:], 512)
    h_self, agg2 = _batch(nodes_batch, adj_p, h)
    return _mm_relu(h_self, agg2, W2[:OUT], W2[OUT:], 512)


# speedup vs baseline: 17.7858x; 1.3658x over previous
import functools

import jax
import jax.numpy as jnp
from jax import lax
from jax.experimental import pallas as pl
from jax.experimental.pallas import tpu as pltpu
from jax.experimental.pallas import tpu_sc as plsc

N = 10000
DEG = 32
D = 128
OUT = 128
B = 4096
S1 = 16
S2 = 4

NC, NS, L = 2, 16, 16
NW = NC * NS
NODES_PER_W = 320
CH_A = 32
NCH_A = NODES_PER_W // CH_A
NBUF_A = 4
B_PER_W = B // NW
CH_C = 8
NCH_C = B_PER_W // CH_C
NBUF_C = 4

_MESH = plsc.VectorSubcoreMesh(
    core_axis_name="c", subcore_axis_name="s", num_cores=NC, num_subcores=NS
)


@functools.partial(
    pl.kernel,
    out_type=jax.ShapeDtypeStruct((N, D), jnp.float32),
    mesh=_MESH,
    scratch_types=[
        pltpu.VMEM((NODES_PER_W * S2,), jnp.int32),
    ]
    + [pltpu.VMEM((CH_A * S2, D), jnp.float32) for _ in range(NBUF_A)]
    + [pltpu.VMEM((NODES_PER_W, D), jnp.float32)]
    + [pltpu.SemaphoreType.DMA for _ in range(NBUF_A)],
)
def _preagg(idx_hbm, feat_hbm, out_hbm, idx_v, r0, r1, r2, r3, ob,
            s0, s1, s2, s3):
    rows = (r0, r1, r2, r3)
    sems = (s0, s1, s2, s3)
    wid = lax.axis_index("s") * NC + lax.axis_index("c")
    base = jnp.minimum(wid * NODES_PER_W, N - NODES_PER_W)
    pltpu.sync_copy(idx_hbm.at[pl.ds(base * S2, NODES_PER_W * S2)], idx_v)

    def issue(ch):
        b = ch % NBUF_A
        src = feat_hbm.at[idx_v.at[pl.ds(ch * CH_A * S2, CH_A * S2)]]
        return pltpu.async_copy(src, rows[b], sems[b])

    cps = {ch: issue(ch) for ch in range(NBUF_A)}
    for ch in range(NCH_A):
        b = ch % NBUF_A
        cps[ch].wait()

        def node(i, c2, _rv=rows[b], _off=ch * CH_A):
            for k in range(D // L):
                s = _rv[i * S2, pl.ds(k * L, L)]
                for p in range(1, S2):
                    s = s + _rv[i * S2 + p, pl.ds(k * L, L)]
                ob[_off + i, pl.ds(k * L, L)] = s * (1.0 / S2)
            return c2

        lax.fori_loop(0, CH_A, node, 0)
        if ch + NBUF_A < NCH_A:
            cps[ch + NBUF_A] = issue(ch + NBUF_A)
    pltpu.sync_copy(ob, out_hbm.at[pl.ds(base, NODES_PER_W)])


@functools.partial(
    pl.kernel,
    out_type=(
        jax.ShapeDtypeStruct((B, OUT), jnp.float32),
        jax.ShapeDtypeStruct((B, OUT), jnp.float32),
    ),
    mesh=_MESH,
    scratch_types=[
        pltpu.VMEM((B_PER_W,), jnp.int32),
        pltpu.VMEM((B_PER_W, 128), jnp.int32),
        pltpu.VMEM((B_PER_W, OUT), jnp.float32),
    ]
    + [pltpu.VMEM((CH_C * S1,), jnp.int32) for _ in range(NBUF_C)]
    + [pltpu.VMEM((CH_C * S1, OUT), jnp.float32) for _ in range(NBUF_C)]
    + [pltpu.VMEM((B_PER_W, OUT), jnp.float32)]
    + [pltpu.SemaphoreType.DMA for _ in range(NBUF_C + 2)],
)
def _batch(nb_hbm, adj_hbm, h_hbm, hself_out, agg_out,
           nb_v, adjr_v, hself_v, i0, i1, i2, i3, r0, r1, r2, r3, aggb,
           s0, s1, s2, s3, sa, sh):
    nidx = (i0, i1, i2, i3)
    nrows = (r0, r1, r2, r3)
    sems = (s0, s1, s2, s3)
    wid = lax.axis_index("s") * NC + lax.axis_index("c")
    base = wid * B_PER_W
    pltpu.sync_copy(nb_hbm.at[pl.ds(base, B_PER_W)], nb_v)
    cp_adj = pltpu.async_copy(adj_hbm.at[nb_v], adjr_v, sa)
    cp_self = pltpu.async_copy(h_hbm.at[nb_v], hself_v, sh)
    cp_adj.wait()

    def issue(ch):
        b = ch % NBUF_C
        _ni = nidx[b]

        def build(j, c2, _off=ch * CH_C):
            _ni[pl.ds(j * S1, S1)] = adjr_v[_off + j, pl.ds(0, S1)]
            return c2

        lax.fori_loop(0, CH_C, build, 0)
        return pltpu.async_copy(h_hbm.at[_ni], nrows[b], sems[b])

    cps = {ch: issue(ch) for ch in range(NBUF_C)}
    for ch in range(NCH_C):
        b = ch % NBUF_C
        cps[ch].wait()

        def bacc(j, c2, _rv=nrows[b], _off=ch * CH_C):
            def kstep(k, c3):
                s = _rv[j * S1, pl.ds(k * L, L)]
                for p in range(1, S1):
                    s = s + _rv[j * S1 + p, pl.ds(k * L, L)]
                aggb[_off + j, pl.ds(k * L, L)] = s * (1.0 / S1)
                return c3

            lax.fori_loop(0, OUT // L, kstep, 0)
            return c2

        lax.fori_loop(0, CH_C, bacc, 0)
        if ch + NBUF_C < NCH_C:
            cps[ch + NBUF_C] = issue(ch + NBUF_C)
    cp_self.wait()
    pltpu.sync_copy(hself_v, hself_out.at[pl.ds(base, B_PER_W)])
    pltpu.sync_copy(aggb, agg_out.at[pl.ds(base, B_PER_W)])


def _mm_body(a_ref, b_ref, wa_ref, wb_ref, o_ref):
    acc = jnp.dot(a_ref[...], wa_ref[...], preferred_element_type=jnp.float32)
    acc = acc + jnp.dot(b_ref[...], wb_ref[...], preferred_element_type=jnp.float32)
    o_ref[...] = jnp.maximum(acc, 0.0)


def _mm_relu(a, b, wa, wb, bm):
    m = a.shape[0]
    return pl.pallas_call(
        _mm_body,
        grid=(m // bm,),
        in_specs=[
            pl.BlockSpec((bm, D), lambda i: (i, 0)),
            pl.BlockSpec((bm, D), lambda i: (i, 0)),
            pl.BlockSpec((D, OUT), lambda i: (0, 0)),
            pl.BlockSpec((D, OUT), lambda i: (0, 0)),
        ],
        out_specs=pl.BlockSpec((bm, OUT), lambda i: (i, 0)),
        out_shape=jax.ShapeDtypeStruct((m, OUT), jnp.float32),
    )(a, b, wa, wb)


def kernel(nodes_batch, adj, raw_features, W1, W2):
    idx_a = adj[:, :S2].reshape(-1)
    adj_p = jnp.pad(adj[:, :S1], ((0, 0), (0, 128 - S1)))
    pre_agg = _preagg(idx_a, raw_features)
    h = _mm_relu(raw_features, pre_agg, W1[:D], W1[D:], 1000)
    h_self, agg2 = _batch(nodes_batch, adj_p, h)
    return _mm_relu(h_self, agg2, W2[:OUT], W2[OUT:], 512)
